# Initial kernel scaffold; baseline (speedup 1.0000x reference)
import jax
import jax.numpy as jnp
from jax.experimental import pallas as pl


def kernel(hidden_states, router_w, gate_w, up_w, down_w):
    raise NotImplementedError("write your pallas kernel here")


# speedup vs baseline: 1.8201x; 1.8201x over previous
import jax
import jax.numpy as jnp
from jax.experimental import pallas as pl
from jax.experimental.pallas import tpu as pltpu

_B, _S, _H, _F, _E, _K = 1, 2048, 1024, 2816, 8, 2
_T = _B * _S
_P = _T * _K
_BLK = 256
_NB = _P // _BLK + _E
_FT = 1408
_NF = _F // _FT


def _router_kernel(hs_ref, rw_ref, logits_ref, w_ref, e_ref):
    hs = hs_ref[...]
    logits = jax.lax.dot_general(hs, rw_ref[...], (((1,), (1,)), ((), ())),
                                 preferred_element_type=jnp.float32)
    logits_ref[...] = logits
    m = jnp.max(logits, axis=1, keepdims=True)
    ex = jnp.exp(logits - m)
    sm = ex / jnp.sum(ex, axis=1, keepdims=True)
    iota = jax.lax.broadcasted_iota(jnp.int32, sm.shape, 1)
    m1 = jnp.max(sm, axis=1, keepdims=True)
    a1 = jnp.min(jnp.where(sm == m1, iota, _E), axis=1, keepdims=True)
    sm2 = jnp.where(iota == a1, -jnp.inf, sm)
    m2 = jnp.max(sm2, axis=1, keepdims=True)
    a2 = jnp.min(jnp.where(sm2 == m2, iota, _E), axis=1, keepdims=True)
    w_ref[...] = jnp.concatenate([m1, m2], axis=1)
    e_ref[...] = jnp.concatenate([a1, a2], axis=1)


def _moe_kernel(be_ref, rt_ref, ba_ref, hs_ref, g_ref, u_ref, d_ref, w_ref,
                out_ref, x_s, acc):
    b = pl.program_id(0)
    f = pl.program_id(1)

    @pl.when((b == 0) & (f == 0))
    def _():
        out_ref[...] = jnp.zeros_like(out_ref)

    active = ba_ref[b] == 1

    @pl.when(active)
    def _():
        @pl.when(f == 0)
        def _():
            def gather(i, c):
                t = rt_ref[b * _BLK + i]
                x_s[pl.ds(i, 1), :] = hs_ref[pl.ds(t, 1), :]
                return c
            jax.lax.fori_loop(0, _BLK, gather, 0, unroll=8)

        x = x_s[...]
        g = jax.lax.dot_general(x, g_ref[0], (((1,), (1,)), ((), ())),
                                preferred_element_type=jnp.float32)
        u = jax.lax.dot_general(x, u_ref[0], (((1,), (1,)), ((), ())),
                                preferred_element_type=jnp.float32)
        h = (g * jax.nn.sigmoid(g)) * u
        part = jax.lax.dot_general(h, d_ref[0], (((1,), (1,)), ((), ())),
                                   preferred_element_type=jnp.float32)

        @pl.when(f == 0)
        def _():
            acc[...] = part

        @pl.when(f != 0)
        def _():
            acc[...] += part

        @pl.when(f == _NF - 1)
        def _():
            x_s[...] = acc[...] * w_ref[...]

            def scat(i, c):
                t = rt_ref[b * _BLK + i]
                out_ref[pl.ds(t, 1), :] += x_s[pl.ds(i, 1), :]
                return c
            jax.lax.fori_loop(0, _BLK, scat, 0, unroll=8)


def kernel(hidden_states, router_w, gate_w, up_w, down_w):
    b, s, h = hidden_states.shape
    hs = hidden_states.reshape(-1, h)

    logits, tw, te = pl.pallas_call(
        _router_kernel,
        out_shape=[
            jax.ShapeDtypeStruct((_T, _E), jnp.float32),
            jax.ShapeDtypeStruct((_T, _K), jnp.float32),
            jax.ShapeDtypeStruct((_T, _K), jnp.int32),
        ],
    )(hs, router_w)

    es = te.reshape(-1)
    ws = tw.reshape(-1)
    onehot = (es[:, None] == jnp.arange(_E)[None, :]).astype(jnp.int32)
    within = jnp.cumsum(onehot, axis=0) - onehot
    rank = jnp.sum(within * onehot, axis=1)
    counts = jnp.sum(onehot, axis=0)
    padded = ((counts + _BLK - 1) // _BLK) * _BLK
    pend = jnp.cumsum(padded)
    poff = pend - padded
    pos = poff[es] + rank
    pp = _NB * _BLK
    row_token = jnp.zeros((pp,), jnp.int32).at[pos].set(
        jnp.arange(_P, dtype=jnp.int32) // _K)
    row_weight = jnp.zeros((pp, 1), jnp.float32).at[pos, 0].set(ws)
    starts = jnp.arange(_NB, dtype=jnp.int32) * _BLK
    total = pend[_E - 1]
    block_active = (starts < total).astype(jnp.int32)
    starts_c = jnp.minimum(starts, total - 1)
    block_expert = jnp.searchsorted(pend, starts_c, side='right').astype(jnp.int32)

    grid_spec = pltpu.PrefetchScalarGridSpec(
        num_scalar_prefetch=3,
        grid=(_NB, _NF),
        in_specs=[
            pl.BlockSpec((_T, _H), lambda bb, ff, be, rt, ba: (0, 0)),
            pl.BlockSpec((1, _FT, _H), lambda bb, ff, be, rt, ba: (be[bb], ff, 0)),
            pl.BlockSpec((1, _FT, _H), lambda bb, ff, be, rt, ba: (be[bb], ff, 0)),
            pl.BlockSpec((1, _H, _FT), lambda bb, ff, be, rt, ba: (be[bb], 0, ff)),
            pl.BlockSpec((_BLK, 1), lambda bb, ff, be, rt, ba: (bb, 0)),
        ],
        out_specs=pl.BlockSpec((_T, _H), lambda bb, ff, be, rt, ba: (0, 0)),
        scratch_shapes=[
            pltpu.VMEM((_BLK, _H), jnp.float32),
            pltpu.VMEM((_BLK, _H), jnp.float32),
        ],
    )

    out = pl.pallas_call(
        _moe_kernel,
        grid_spec=grid_spec,
        out_shape=jax.ShapeDtypeStruct((_T, _H), jnp.float32),
        compiler_params=pltpu.CompilerParams(
            dimension_semantics=("arbitrary", "arbitrary"),
        ),
    )(block_expert, row_token, block_active,
      hs, gate_w, up_w, down_w, row_weight)

    return out.reshape(b, s, h), logits
